# baseline (speedup 1.0000x reference)
import jax
import jax.numpy as jnp
from jax.experimental import pallas as pl
from jax.experimental.pallas import tpu as pltpu

B, S, H = 4, 8192, 768
TYPE_VOCAB = 2
PAD_OFF = 2
EPS = 1e-12

SBLK = 1024
NS = S // SBLK


def _embed_ln_kernel(x_ref, posa_ref, posb_ref, tt_ref, g_ref, b_ref, o_ref):
    pos = jnp.concatenate([posa_ref[PAD_OFF:, :], posb_ref[:PAD_OFF, :]], axis=0)
    bias = pos + tt_ref[0:1, :]
    for b in range(B):
        y = x_ref[b] + bias
        mean = jnp.mean(y, axis=-1, keepdims=True)
        var = jnp.mean(y * y, axis=-1, keepdims=True) - mean * mean
        o_ref[b] = (y - mean) * jax.lax.rsqrt(var + EPS) * g_ref[...] + b_ref[...]


def kernel(inputs_embeds, token_type_table, position_table, ln_gamma, ln_beta):
    g2 = ln_gamma.reshape(1, H)
    b2 = ln_beta.reshape(1, H)
    return pl.pallas_call(
        _embed_ln_kernel,
        grid=(NS,),
        in_specs=[
            pl.BlockSpec((B, SBLK, H), lambda i: (0, i, 0)),
            pl.BlockSpec((SBLK, H), lambda i: (i, 0)),
            pl.BlockSpec((8, H), lambda i: ((SBLK // 8) * (i + 1), 0)),
            pl.BlockSpec((TYPE_VOCAB, H), lambda i: (0, 0)),
            pl.BlockSpec((1, H), lambda i: (0, 0)),
            pl.BlockSpec((1, H), lambda i: (0, 0)),
        ],
        out_specs=pl.BlockSpec((B, SBLK, H), lambda i: (0, i, 0)),
        out_shape=jax.ShapeDtypeStruct((B, S, H), jnp.float32),
        compiler_params=pltpu.CompilerParams(
            dimension_semantics=("parallel",),
            vmem_limit_bytes=64 * 1024 * 1024,
        ),
    )(inputs_embeds, position_table, position_table, token_type_table, g2, b2)


# speedup vs baseline: 1.0089x; 1.0089x over previous
import jax
import jax.numpy as jnp
from jax.experimental import pallas as pl
from jax.experimental.pallas import tpu as pltpu

B, S, H = 4, 8192, 768
TYPE_VOCAB = 2
PAD_OFF = 2
EPS = 1e-12

SBLK = 512
NS = S // SBLK


def _embed_ln_kernel(x_ref, posa_ref, posb_ref, tt_ref, g_ref, b_ref, o_ref):
    pos = jnp.concatenate([posa_ref[PAD_OFF:, :], posb_ref[:PAD_OFF, :]], axis=0)
    bias = pos + tt_ref[0:1, :]
    for b in range(B):
        y = x_ref[b] + bias
        mean = jnp.mean(y, axis=-1, keepdims=True)
        var = jnp.mean(y * y, axis=-1, keepdims=True) - mean * mean
        o_ref[b] = (y - mean) * jax.lax.rsqrt(var + EPS) * g_ref[...] + b_ref[...]


def kernel(inputs_embeds, token_type_table, position_table, ln_gamma, ln_beta):
    g2 = ln_gamma.reshape(1, H)
    b2 = ln_beta.reshape(1, H)
    return pl.pallas_call(
        _embed_ln_kernel,
        grid=(NS,),
        in_specs=[
            pl.BlockSpec((B, SBLK, H), lambda i: (0, i, 0)),
            pl.BlockSpec((SBLK, H), lambda i: (i, 0)),
            pl.BlockSpec((8, H), lambda i: ((SBLK // 8) * (i + 1), 0)),
            pl.BlockSpec((TYPE_VOCAB, H), lambda i: (0, 0)),
            pl.BlockSpec((1, H), lambda i: (0, 0)),
            pl.BlockSpec((1, H), lambda i: (0, 0)),
        ],
        out_specs=pl.BlockSpec((B, SBLK, H), lambda i: (0, i, 0)),
        out_shape=jax.ShapeDtypeStruct((B, S, H), jnp.float32),
        compiler_params=pltpu.CompilerParams(
            dimension_semantics=("parallel",),
        ),
    )(inputs_embeds, position_table, position_table, token_type_table, g2, b2)
